# baseline (speedup 1.0000x reference)
import functools

import jax
import jax.numpy as jnp
from jax import lax
from jax.experimental import pallas as pl
from jax.experimental.pallas import tpu as pltpu
from jax.experimental.pallas import tpu_sc as plsc

BATCH = 16384
IN_F = 512
OUT_F = 64

NC = 2
NS = 16
L = 16
NW = NC * NS
ROWS_W = BATCH // NW
CHUNK = 64
NCHUNK = ROWS_W // CHUNK
NG = OUT_F // L


def _sc_gather(x_flat, columns):
    mesh = plsc.VectorSubcoreMesh(core_axis_name="c", subcore_axis_name="s")

    @functools.partial(
        pl.kernel,
        mesh=mesh,
        out_type=jax.ShapeDtypeStruct((BATCH * OUT_F,), jnp.float32),
        compiler_params=pltpu.CompilerParams(needs_layout_passes=False),
        scratch_types=[
            pltpu.VMEM((OUT_F,), jnp.int32),
            pltpu.VMEM((CHUNK * IN_F,), jnp.float32),
            pltpu.VMEM((CHUNK * OUT_F,), jnp.float32),
        ],
    )
    def k(x_hbm, cols_hbm, out_hbm, cols_v, in_v, out_v):
        wid = lax.axis_index("s") * NC + lax.axis_index("c")
        base = wid * ROWS_W
        pltpu.sync_copy(cols_hbm, cols_v)
        col_regs = [cols_v[pl.ds(g * L, L)] for g in range(NG)]
        out_cols = [lax.iota(jnp.int32, L) + g * L for g in range(NG)]

        def chunk_body(ci, _):
            row0 = base + ci * CHUNK
            pltpu.sync_copy(x_hbm.at[pl.ds(row0 * IN_F, CHUNK * IN_F)], in_v)

            def row_body(r, _):
                in_off = r * IN_F
                out_off = r * OUT_F
                for g in range(NG):
                    vals = plsc.load_gather(in_v, [col_regs[g] + in_off])
                    plsc.store_scatter(out_v, [out_cols[g] + out_off], vals)
                return 0

            lax.fori_loop(0, CHUNK, row_body, 0)
            pltpu.sync_copy(out_v, out_hbm.at[pl.ds(row0 * OUT_F, CHUNK * OUT_F)])
            return 0

        lax.fori_loop(0, NCHUNK, chunk_body, 0)

    return k(x_flat, columns)


def kernel(x, columns):
    out_flat = _sc_gather(x.reshape(-1), columns)
    return out_flat.reshape(BATCH, OUT_F)


# speedup vs baseline: 1.0974x; 1.0974x over previous
import functools

import jax
import jax.numpy as jnp
from jax import lax
from jax.experimental import pallas as pl
from jax.experimental.pallas import tpu as pltpu
from jax.experimental.pallas import tpu_sc as plsc

BATCH = 16384
IN_F = 512
OUT_F = 64

NC = 2
NS = 16
L = 16
NW = NC * NS
ROWS_W = BATCH // NW
CHUNK = 64
NCHUNK = ROWS_W // CHUNK
NG = OUT_F // L


def _sc_gather(x_flat, columns):
    mesh = plsc.VectorSubcoreMesh(core_axis_name="c", subcore_axis_name="s")

    @functools.partial(
        pl.kernel,
        mesh=mesh,
        out_type=jax.ShapeDtypeStruct((BATCH * OUT_F,), jnp.float32),
        compiler_params=pltpu.CompilerParams(needs_layout_passes=False),
        scratch_types=[
            pltpu.VMEM((OUT_F,), jnp.int32),
            pltpu.VMEM((CHUNK * IN_F,), jnp.float32),
            pltpu.VMEM((CHUNK * IN_F,), jnp.float32),
            pltpu.VMEM((CHUNK * OUT_F,), jnp.float32),
            pltpu.VMEM((CHUNK * OUT_F,), jnp.float32),
            pltpu.SemaphoreType.DMA,
            pltpu.SemaphoreType.DMA,
            pltpu.SemaphoreType.DMA,
            pltpu.SemaphoreType.DMA,
        ],
    )
    def k(x_hbm, cols_hbm, out_hbm, cols_v, in0, in1, ou0, ou1, is0, is1, os0, os1):
        wid = lax.axis_index("s") * NC + lax.axis_index("c")
        base = wid * ROWS_W
        pltpu.sync_copy(cols_hbm, cols_v)
        col_regs = [cols_v[pl.ds(g * L, L)] for g in range(NG)]
        ins = [in0, in1]
        outs = [ou0, ou1]
        isem = [is0, is1]
        osem = [os0, os1]

        def start_load(ci):
            row0 = base + ci * CHUNK
            return pltpu.async_copy(
                x_hbm.at[pl.ds(row0 * IN_F, CHUNK * IN_F)], ins[ci % 2], isem[ci % 2]
            )

        loads = [None] * NCHUNK
        stores = [None] * NCHUNK
        loads[0] = start_load(0)
        for ci in range(NCHUNK):
            if ci + 1 < NCHUNK:
                loads[ci + 1] = start_load(ci + 1)
            loads[ci].wait()
            if ci >= 2:
                stores[ci - 2].wait()
            ib = ins[ci % 2]
            ob = outs[ci % 2]

            @plsc.parallel_loop(0, CHUNK, unroll=4)
            def row_body(r):
                in_off = r * IN_F
                out_off = r * OUT_F
                for g in range(NG):
                    vals = plsc.load_gather(ib, [col_regs[g] + in_off])
                    ob[pl.ds(out_off + g * L, L)] = vals

            row0 = base + ci * CHUNK
            stores[ci] = pltpu.async_copy(
                ob, out_hbm.at[pl.ds(row0 * OUT_F, CHUNK * OUT_F)], osem[ci % 2]
            )
        stores[NCHUNK - 2].wait()
        stores[NCHUNK - 1].wait()

    return k(x_flat, columns)


def kernel(x, columns):
    out_flat = _sc_gather(x.reshape(-1), columns)
    return out_flat.reshape(BATCH, OUT_F)


# speedup vs baseline: 1.9611x; 1.7870x over previous
import functools

import jax
import jax.numpy as jnp
from jax import lax
from jax.experimental import pallas as pl
from jax.experimental.pallas import tpu as pltpu
from jax.experimental.pallas import tpu_sc as plsc

BATCH = 16384
IN_F = 512
OUT_F = 64

NC = 2
NS = 16
L = 16
NW = NC * NS
ROWS_W = BATCH // NW
CHUNK = 64
NCHUNK = ROWS_W // CHUNK
NG = OUT_F // L


def _sc_gather(x, columns):
    mesh = plsc.VectorSubcoreMesh(core_axis_name="c", subcore_axis_name="s")

    @functools.partial(
        pl.kernel,
        mesh=mesh,
        out_type=jax.ShapeDtypeStruct((BATCH, OUT_F), jnp.float32),
        compiler_params=pltpu.CompilerParams(needs_layout_passes=False),
        scratch_types=[
            pltpu.VMEM((OUT_F,), jnp.int32),
            pltpu.VMEM((CHUNK, IN_F), jnp.float32),
            pltpu.VMEM((CHUNK, IN_F), jnp.float32),
            pltpu.VMEM((CHUNK, OUT_F), jnp.float32),
            pltpu.VMEM((CHUNK, OUT_F), jnp.float32),
            pltpu.SemaphoreType.DMA,
            pltpu.SemaphoreType.DMA,
            pltpu.SemaphoreType.DMA,
            pltpu.SemaphoreType.DMA,
        ],
    )
    def k(x_hbm, cols_hbm, out_hbm, cols_v, in0, in1, ou0, ou1, is0, is1, os0, os1):
        wid = lax.axis_index("s") * NC + lax.axis_index("c")
        base = wid * ROWS_W
        pltpu.sync_copy(cols_hbm, cols_v)
        col_regs = [cols_v[pl.ds(g * L, L)] for g in range(NG)]
        ins = [in0, in1]
        outs = [ou0, ou1]
        isem = [is0, is1]
        osem = [os0, os1]

        def start_load(ci):
            row0 = base + ci * CHUNK
            return pltpu.async_copy(
                x_hbm.at[pl.ds(row0, CHUNK)], ins[ci % 2], isem[ci % 2]
            )

        loads = [None] * NCHUNK
        stores = [None] * NCHUNK
        loads[0] = start_load(0)
        for ci in range(NCHUNK):
            if ci + 1 < NCHUNK:
                loads[ci + 1] = start_load(ci + 1)
            loads[ci].wait()
            if ci >= 2:
                stores[ci - 2].wait()
            ib = ins[ci % 2]
            ob = outs[ci % 2]

            @plsc.parallel_loop(0, CHUNK, unroll=4)
            def row_body(r):
                ridx = jnp.zeros((L,), jnp.int32) + r
                for g in range(NG):
                    vals = plsc.load_gather(ib, [ridx, col_regs[g]])
                    ob[r, pl.ds(g * L, L)] = vals

            row0 = base + ci * CHUNK
            stores[ci] = pltpu.async_copy(
                ob, out_hbm.at[pl.ds(row0, CHUNK)], osem[ci % 2]
            )
        stores[NCHUNK - 2].wait()
        stores[NCHUNK - 1].wait()

    return k(x, columns)


def kernel(x, columns):
    return _sc_gather(x, columns)
